# baseline (speedup 1.0000x reference)
import functools

import jax
import jax.numpy as jnp
from jax import lax
from jax.experimental import pallas as pl
from jax.experimental.pallas import tpu as pltpu
from jax.experimental.pallas import tpu_sc as plsc

_D = 128
_L = 50
_LH = 56
_E = 8
_EPS = 1e-5
_CAP = 32
_GB = 128

_NC, _NS = 2, 16
_NW = _NC * _NS


def _sc_gather(table, idx, chunk):
    m, d = idx.shape[0], table.shape[1]
    per_w = m // _NW
    n_chunks = per_w // chunk
    mesh = plsc.VectorSubcoreMesh(
        core_axis_name="c", subcore_axis_name="s",
        num_cores=_NC, num_subcores=_NS)

    @functools.partial(
        pl.kernel, mesh=mesh,
        out_type=jax.ShapeDtypeStruct((m, d), table.dtype),
        scratch_types=[
            pltpu.VMEM((chunk,), jnp.int32),
            pltpu.VMEM((chunk,), jnp.int32),
            pltpu.VMEM((chunk, d), table.dtype),
            pltpu.VMEM((chunk, d), table.dtype),
            pltpu.SemaphoreType.DMA,
            pltpu.SemaphoreType.DMA,
        ],
    )
    def k(table_hbm, idx_hbm, out_hbm, idx_a, idx_b, rows_a, rows_b,
          sem_a, sem_b):
        wid = lax.axis_index("s") * _NC + lax.axis_index("c")
        base = wid * per_w
        idx_v, rows_v, sems = (idx_a, idx_b), (rows_a, rows_b), (sem_a, sem_b)
        descs = [None, None]

        pltpu.sync_copy(idx_hbm.at[pl.ds(base, chunk)], idx_v[0])
        descs[0] = pltpu.async_copy(table_hbm.at[idx_v[0]], rows_v[0],
                                    sems[0])
        for c in range(n_chunks):
            b, nb = c % 2, (c + 1) % 2
            if c + 1 < n_chunks:
                off = base + (c + 1) * chunk
                pltpu.sync_copy(idx_hbm.at[pl.ds(off, chunk)], idx_v[nb])
                descs[nb] = pltpu.async_copy(table_hbm.at[idx_v[nb]],
                                             rows_v[nb], sems[nb])
            descs[b].wait()
            pltpu.sync_copy(rows_v[b], out_hbm.at[pl.ds(base + c * chunk,
                                                        chunk)])

    return k(table, idx)


def _tap_sum(x2, w_ref, ts0, ts2, n, cout):
    t0 = jnp.dot(x2, w_ref[0, 0], preferred_element_type=jnp.float32)
    t1 = jnp.dot(x2, w_ref[0, 1], preferred_element_type=jnp.float32)
    t2 = jnp.dot(x2, w_ref[0, 2], preferred_element_type=jnp.float32)
    ts0[pl.ds(8, n), pl.ds(0, cout)] = t0
    ts2[pl.ds(8, n), pl.ds(0, cout)] = t2
    return (ts0[pl.ds(7, n), pl.ds(0, cout)] + t1) \
        + ts2[pl.ds(9, n), pl.ds(0, cout)]


def _halo_mask(n):
    pos = lax.broadcasted_iota(jnp.int32, (n, 1), 0) % _LH
    return jnp.logical_and(pos >= 1, pos <= _L)


def _gate_route(embh, gw, cb, g, be, w1t, b1, w2t, b2):
    b_total = embh.shape[0] // _LH
    nb = b_total // _GB
    n = _GB * _LH

    def body(emb_ref, gw_ref, cb_ref, g_ref, be_ref, w1_ref, b1_ref, w2_ref,
             b2_ref, out_ref, ts0, ts2):
        xin = jnp.where(_halo_mask(n), emb_ref[...], 0.0)
        y = _tap_sum(xin, gw_ref, ts0, ts2, n, 64)
        y = y + cb_ref[...]
        y = y / jnp.sqrt(jnp.float32(1.0 + _EPS)) * g_ref[...] + be_ref[...]
        h = jnp.where(_halo_mask(n), jnp.maximum(y, 0.0), 0.0)
        hm = jnp.max(h.reshape(_GB, _LH, 64), axis=1)
        h2 = jnp.maximum(
            jnp.dot(hm, w1_ref[...], preferred_element_type=jnp.float32)
            + b1_ref[...], 0.0)
        lg = jnp.dot(h2, w2_ref[...], preferred_element_type=jnp.float32) \
            + b2_ref[...]
        pm = jnp.exp(lg - jnp.max(lg, axis=1, keepdims=True))
        p = pm / jnp.sum(pm, axis=1, keepdims=True)
        mx = jnp.max(p, axis=1, keepdims=True)
        cand = jnp.where(p >= mx,
                         lax.broadcasted_iota(jnp.int32, p.shape, 1), _E)
        out_ref[0, 0] = jnp.min(cand, axis=1)

    out = pl.pallas_call(
        body,
        grid=(nb,),
        in_specs=[
            pl.BlockSpec((n, _D), lambda i: (i, 0)),
            pl.BlockSpec((1, 3, _D, 64), lambda i: (0, 0, 0, 0)),
            pl.BlockSpec((1, 64), lambda i: (0, 0)),
            pl.BlockSpec((1, 64), lambda i: (0, 0)),
            pl.BlockSpec((1, 64), lambda i: (0, 0)),
            pl.BlockSpec((64, 32), lambda i: (0, 0)),
            pl.BlockSpec((1, 32), lambda i: (0, 0)),
            pl.BlockSpec((32, _E), lambda i: (0, 0)),
            pl.BlockSpec((1, _E), lambda i: (0, 0)),
        ],
        out_specs=pl.BlockSpec((1, 1, _GB), lambda i: (i, 0, 0)),
        out_shape=jax.ShapeDtypeStruct((nb, 1, _GB), jnp.int32),
        scratch_shapes=[pltpu.VMEM((n + 16, 64), jnp.float32),
                        pltpu.VMEM((n + 16, 64), jnp.float32)],
    )(embh, gw, cb, g, be, w1t, b1, w2t, b2)
    return out.reshape(b_total)


def _expert_blocks(emb_p, bexp, nused, w1, bb1, w2, bb2, w3, bb3,
                   f1w, f1b, f2w, f2b):
    nblk = emb_p.shape[0] // (_CAP * _LH)
    n = _CAP * _LH

    def body(bexp_ref, nused_ref, emb_ref, w1_ref, b1_ref, w2_ref,
             b2_ref, w3_ref, b3_ref, f1w_ref, f1b_ref, f2w_ref, f2b_ref,
             out_ref, ts0, ts2):
        gidx = pl.program_id(0)

        @pl.when(gidx < nused_ref[0])
        def _():
            hmask = _halo_mask(n)
            xin = jnp.where(hmask, emb_ref[...], 0.0)
            y1 = _tap_sum(xin, w1_ref, ts0, ts2, n, 64)
            h1 = jnp.where(hmask, jnp.maximum(y1 + b1_ref[0], 0.0), 0.0)
            y2 = _tap_sum(h1, w2_ref, ts0, ts2, n, 32)
            h2 = jnp.where(hmask, jnp.maximum(y2 + b2_ref[0], 0.0), 0.0)
            y3 = _tap_sum(h2, w3_ref, ts0, ts2, n, 16)
            h3 = jnp.where(hmask, jnp.maximum(y3 + b3_ref[0], 0.0), 0.0)
            m = jnp.max(h3.reshape(_CAP, _LH, 16), axis=1)
            f = jnp.maximum(
                jnp.dot(m, f1w_ref[...], preferred_element_type=jnp.float32)
                + f1b_ref[...], 0.0)
            o = jnp.dot(
                f, f2w_ref[...], preferred_element_type=jnp.float32) \
                + f2b_ref[...]
            out_ref[0, :, :16] = o

    grid_spec = pltpu.PrefetchScalarGridSpec(
        num_scalar_prefetch=2,
        grid=(nblk,),
        in_specs=[
            pl.BlockSpec((n, _D), lambda gi, bexp, nu: (gi, 0)),
            pl.BlockSpec((1, 3, _D, 64),
                         lambda gi, bexp, nu: (bexp[gi], 0, 0, 0)),
            pl.BlockSpec((1, 1, 64), lambda gi, bexp, nu: (bexp[gi], 0, 0)),
            pl.BlockSpec((1, 3, 64, 32),
                         lambda gi, bexp, nu: (bexp[gi], 0, 0, 0)),
            pl.BlockSpec((1, 1, 32), lambda gi, bexp, nu: (bexp[gi], 0, 0)),
            pl.BlockSpec((1, 3, 32, 16),
                         lambda gi, bexp, nu: (bexp[gi], 0, 0, 0)),
            pl.BlockSpec((1, 1, 16), lambda gi, bexp, nu: (bexp[gi], 0, 0)),
            pl.BlockSpec((16, 64), lambda gi, bexp, nu: (0, 0)),
            pl.BlockSpec((1, 64), lambda gi, bexp, nu: (0, 0)),
            pl.BlockSpec((64, 16), lambda gi, bexp, nu: (0, 0)),
            pl.BlockSpec((1, 16), lambda gi, bexp, nu: (0, 0)),
        ],
        out_specs=pl.BlockSpec((1, _CAP, 128),
                               lambda gi, bexp, nu: (gi, 0, 0)),
        scratch_shapes=[pltpu.VMEM((n + 16, 64), jnp.float32),
                        pltpu.VMEM((n + 16, 64), jnp.float32)],
    )
    return pl.pallas_call(
        body,
        grid_spec=grid_spec,
        out_shape=jax.ShapeDtypeStruct((nblk, _CAP, 128), jnp.float32),
    )(bexp, nused, emb_p, w1, bb1, w2, bb2, w3, bb3, f1w, f1b, f2w, f2b)


def _fold_conv(cw, cb, g, be):
    s = g / jnp.sqrt(1.0 + _EPS)
    w = jnp.transpose(cw * s[:, None, None], (2, 1, 0))
    b = (cb * s + be)[None, :]
    return w, b


def _run_half(x, params, wts):
    b_total = x.shape[0]
    gw, ws, f2w, f2b = wts

    xi = x.astype(jnp.int32)
    idx1 = jnp.concatenate(
        [xi[:, :1], xi, jnp.broadcast_to(xi[:, -1:], (b_total, 5))], axis=1)
    embh = _sc_gather(params["embedding"], idx1.reshape(b_total * _LH), 448)

    gp = params["gate"]
    top_idx = _gate_route(
        embh, gw[None], gp["cb"][None, :], gp["g"][None, :],
        gp["be"][None, :], gp["w1"].T, gp["b1"][None, :],
        gp["w2"].T, gp["b2"][None, :])

    nblk = b_total // _CAP + _E
    p = nblk * _CAP
    oh = (top_idx[:, None] == jnp.arange(_E, dtype=jnp.int32)).astype(jnp.int32)
    rank = jnp.sum((jnp.cumsum(oh, axis=0) - oh) * oh, axis=1)
    counts = jnp.sum(oh, axis=0)
    blocks_e = (counts + _CAP - 1) // _CAP
    start_blk = jnp.concatenate(
        [jnp.zeros((1,), jnp.int32), jnp.cumsum(blocks_e)[:-1]])
    dest = start_blk[top_idx] * _CAP + rank
    src = jnp.zeros((p,), jnp.int32).at[dest].set(
        jnp.arange(b_total, dtype=jnp.int32))
    gids = jnp.arange(nblk, dtype=jnp.int32)
    bexp = jnp.sum(gids[:, None] >= start_blk[None, :], axis=1,
                   dtype=jnp.int32) - 1
    nused = jnp.sum(blocks_e, dtype=jnp.int32)[None]

    xp_idx = (src[:, None] * _LH
              + jnp.arange(_LH, dtype=jnp.int32)[None, :]).reshape(p * _LH)
    emb_p = _sc_gather(embh, xp_idx, 448)
    out_p = _expert_blocks(
        emb_p, bexp, nused, *ws,
        params["fc1_w"].T, params["fc1_b"][None, :], f2w, f2b)

    out_rows = _sc_gather(out_p.reshape(p, 128), dest, b_total // _NW)
    return out_rows[:, :2]


def kernel(x, params):
    gp = params["gate"]
    gw = jnp.transpose(gp["cw"], (2, 1, 0))
    ws1, ws2, ws3, bs1, bs2, bs3 = [], [], [], [], [], []
    for ep in params["experts"]:
        w1, b1 = _fold_conv(ep["w1"], ep["b1"], ep["g1"], ep["be1"])
        w2, b2 = _fold_conv(ep["w2"], ep["b2"], ep["g2"], ep["be2"])
        w3, b3 = _fold_conv(ep["w3"], ep["b3"], ep["g3"], ep["be3"])
        ws1.append(w1); ws2.append(w2); ws3.append(w3)
        bs1.append(b1); bs2.append(b2); bs3.append(b3)
    f2w = jnp.zeros((64, 16), jnp.float32).at[:, :2].set(params["fc2_w"].T)
    f2b = jnp.zeros((1, 16), jnp.float32).at[:, :2].set(params["fc2_b"][None])
    ws = (jnp.stack(ws1), jnp.stack(bs1), jnp.stack(ws2), jnp.stack(bs2),
          jnp.stack(ws3), jnp.stack(bs3))
    wts = (gw, ws, f2w, f2b)
    q = x.shape[0] // 4
    outs = [_run_half(x[i * q:(i + 1) * q], params, wts) for i in range(4)]
    return jnp.concatenate(outs, axis=0)


# speedup vs baseline: 1.1114x; 1.1114x over previous
import functools

import jax
import jax.numpy as jnp
from jax import lax
from jax.experimental import pallas as pl
from jax.experimental.pallas import tpu as pltpu
from jax.experimental.pallas import tpu_sc as plsc

_D = 128
_L = 50
_LH = 56
_E = 8
_EPS = 1e-5
_CAP = 32
_GB = 256

_NC, _NS = 2, 16
_NW = _NC * _NS


def _sc_gather(table, idx, chunk):
    m, d = idx.shape[0], table.shape[1]
    per_w = m // _NW
    n_chunks = per_w // chunk
    mesh = plsc.VectorSubcoreMesh(
        core_axis_name="c", subcore_axis_name="s",
        num_cores=_NC, num_subcores=_NS)

    @functools.partial(
        pl.kernel, mesh=mesh,
        out_type=jax.ShapeDtypeStruct((m, d), table.dtype),
        scratch_types=[
            pltpu.VMEM((chunk,), jnp.int32),
            pltpu.VMEM((chunk,), jnp.int32),
            pltpu.VMEM((chunk, d), table.dtype),
            pltpu.VMEM((chunk, d), table.dtype),
            pltpu.SemaphoreType.DMA,
            pltpu.SemaphoreType.DMA,
        ],
    )
    def k(table_hbm, idx_hbm, out_hbm, idx_a, idx_b, rows_a, rows_b,
          sem_a, sem_b):
        wid = lax.axis_index("s") * _NC + lax.axis_index("c")
        base = wid * per_w
        idx_v, rows_v, sems = (idx_a, idx_b), (rows_a, rows_b), (sem_a, sem_b)
        descs = [None, None]

        pltpu.sync_copy(idx_hbm.at[pl.ds(base, chunk)], idx_v[0])
        descs[0] = pltpu.async_copy(table_hbm.at[idx_v[0]], rows_v[0],
                                    sems[0])
        for c in range(n_chunks):
            b, nb = c % 2, (c + 1) % 2
            if c + 1 < n_chunks:
                off = base + (c + 1) * chunk
                pltpu.sync_copy(idx_hbm.at[pl.ds(off, chunk)], idx_v[nb])
                descs[nb] = pltpu.async_copy(table_hbm.at[idx_v[nb]],
                                             rows_v[nb], sems[nb])
            descs[b].wait()
            pltpu.sync_copy(rows_v[b], out_hbm.at[pl.ds(base + c * chunk,
                                                        chunk)])

    return k(table, idx)


def _tap_sum(x2, w_ref, ts0, ts2, n, cout):
    t0 = jnp.dot(x2, w_ref[0, 0], preferred_element_type=jnp.float32)
    t1 = jnp.dot(x2, w_ref[0, 1], preferred_element_type=jnp.float32)
    t2 = jnp.dot(x2, w_ref[0, 2], preferred_element_type=jnp.float32)
    ts0[pl.ds(8, n), pl.ds(0, cout)] = t0
    ts2[pl.ds(8, n), pl.ds(0, cout)] = t2
    return (ts0[pl.ds(7, n), pl.ds(0, cout)] + t1) \
        + ts2[pl.ds(9, n), pl.ds(0, cout)]


def _halo_mask(n):
    pos = lax.broadcasted_iota(jnp.int32, (n, 1), 0) % _LH
    return jnp.logical_and(pos >= 1, pos <= _L)


def _gate_route(embh, gw, cb, g, be, w1t, b1, w2t, b2):
    b_total = embh.shape[0] // _LH
    nb = b_total // _GB
    n = _GB * _LH

    def body(emb_ref, gw_ref, cb_ref, g_ref, be_ref, w1_ref, b1_ref, w2_ref,
             b2_ref, out_ref, ts0, ts2):
        xin = jnp.where(_halo_mask(n), emb_ref[...], 0.0)
        y = _tap_sum(xin, gw_ref, ts0, ts2, n, 64)
        y = y + cb_ref[...]
        y = y / jnp.sqrt(jnp.float32(1.0 + _EPS)) * g_ref[...] + be_ref[...]
        h = jnp.where(_halo_mask(n), jnp.maximum(y, 0.0), 0.0)
        hm = jnp.max(h.reshape(_GB, _LH, 64), axis=1)
        h2 = jnp.maximum(
            jnp.dot(hm, w1_ref[...], preferred_element_type=jnp.float32)
            + b1_ref[...], 0.0)
        lg = jnp.dot(h2, w2_ref[...], preferred_element_type=jnp.float32) \
            + b2_ref[...]
        pm = jnp.exp(lg - jnp.max(lg, axis=1, keepdims=True))
        p = pm / jnp.sum(pm, axis=1, keepdims=True)
        mx = jnp.max(p, axis=1, keepdims=True)
        cand = jnp.where(p >= mx,
                         lax.broadcasted_iota(jnp.int32, p.shape, 1), _E)
        out_ref[0, 0] = jnp.min(cand, axis=1)

    out = pl.pallas_call(
        body,
        grid=(nb,),
        in_specs=[
            pl.BlockSpec((n, _D), lambda i: (i, 0)),
            pl.BlockSpec((1, 3, _D, 64), lambda i: (0, 0, 0, 0)),
            pl.BlockSpec((1, 64), lambda i: (0, 0)),
            pl.BlockSpec((1, 64), lambda i: (0, 0)),
            pl.BlockSpec((1, 64), lambda i: (0, 0)),
            pl.BlockSpec((64, 32), lambda i: (0, 0)),
            pl.BlockSpec((1, 32), lambda i: (0, 0)),
            pl.BlockSpec((32, _E), lambda i: (0, 0)),
            pl.BlockSpec((1, _E), lambda i: (0, 0)),
        ],
        out_specs=pl.BlockSpec((1, 1, _GB), lambda i: (i, 0, 0)),
        out_shape=jax.ShapeDtypeStruct((nb, 1, _GB), jnp.int32),
        scratch_shapes=[pltpu.VMEM((n + 16, 64), jnp.float32),
                        pltpu.VMEM((n + 16, 64), jnp.float32)],
    )(embh, gw, cb, g, be, w1t, b1, w2t, b2)
    return out.reshape(b_total)


def _expert_blocks(emb_p, bexp, nused, w1, bb1, w2, bb2, w3, bb3,
                   f1w, f1b, f2w, f2b):
    nblk = emb_p.shape[0] // (_CAP * _LH)
    n = _CAP * _LH

    def body(bexp_ref, nused_ref, emb_ref, w1_ref, b1_ref, w2_ref,
             b2_ref, w3_ref, b3_ref, f1w_ref, f1b_ref, f2w_ref, f2b_ref,
             out_ref, ts0, ts2):
        gidx = pl.program_id(0)

        @pl.when(gidx < nused_ref[0])
        def _():
            hmask = _halo_mask(n)
            xin = jnp.where(hmask, emb_ref[...], 0.0)
            y1 = _tap_sum(xin, w1_ref, ts0, ts2, n, 64)
            h1 = jnp.where(hmask, jnp.maximum(y1 + b1_ref[0], 0.0), 0.0)
            y2 = _tap_sum(h1, w2_ref, ts0, ts2, n, 32)
            h2 = jnp.where(hmask, jnp.maximum(y2 + b2_ref[0], 0.0), 0.0)
            y3 = _tap_sum(h2, w3_ref, ts0, ts2, n, 16)
            h3 = jnp.where(hmask, jnp.maximum(y3 + b3_ref[0], 0.0), 0.0)
            m = jnp.max(h3.reshape(_CAP, _LH, 16), axis=1)
            f = jnp.maximum(
                jnp.dot(m, f1w_ref[...], preferred_element_type=jnp.float32)
                + f1b_ref[...], 0.0)
            o = jnp.dot(
                f, f2w_ref[...], preferred_element_type=jnp.float32) \
                + f2b_ref[...]
            out_ref[0, :, :16] = o

    grid_spec = pltpu.PrefetchScalarGridSpec(
        num_scalar_prefetch=2,
        grid=(nblk,),
        in_specs=[
            pl.BlockSpec((n, _D), lambda gi, bexp, nu: (gi, 0)),
            pl.BlockSpec((1, 3, _D, 64),
                         lambda gi, bexp, nu: (bexp[gi], 0, 0, 0)),
            pl.BlockSpec((1, 1, 64), lambda gi, bexp, nu: (bexp[gi], 0, 0)),
            pl.BlockSpec((1, 3, 64, 32),
                         lambda gi, bexp, nu: (bexp[gi], 0, 0, 0)),
            pl.BlockSpec((1, 1, 32), lambda gi, bexp, nu: (bexp[gi], 0, 0)),
            pl.BlockSpec((1, 3, 32, 16),
                         lambda gi, bexp, nu: (bexp[gi], 0, 0, 0)),
            pl.BlockSpec((1, 1, 16), lambda gi, bexp, nu: (bexp[gi], 0, 0)),
            pl.BlockSpec((16, 64), lambda gi, bexp, nu: (0, 0)),
            pl.BlockSpec((1, 64), lambda gi, bexp, nu: (0, 0)),
            pl.BlockSpec((64, 16), lambda gi, bexp, nu: (0, 0)),
            pl.BlockSpec((1, 16), lambda gi, bexp, nu: (0, 0)),
        ],
        out_specs=pl.BlockSpec((1, _CAP, 128),
                               lambda gi, bexp, nu: (gi, 0, 0)),
        scratch_shapes=[pltpu.VMEM((n + 16, 64), jnp.float32),
                        pltpu.VMEM((n + 16, 64), jnp.float32)],
    )
    return pl.pallas_call(
        body,
        grid_spec=grid_spec,
        out_shape=jax.ShapeDtypeStruct((nblk, _CAP, 128), jnp.float32),
    )(bexp, nused, emb_p, w1, bb1, w2, bb2, w3, bb3, f1w, f1b, f2w, f2b)


def _fold_conv(cw, cb, g, be):
    s = g / jnp.sqrt(1.0 + _EPS)
    w = jnp.transpose(cw * s[:, None, None], (2, 1, 0))
    b = (cb * s + be)[None, :]
    return w, b


def _run_half(x, params, wts):
    b_total = x.shape[0]
    gw, ws, f2w, f2b = wts

    xi = x.astype(jnp.int32)
    idx1 = jnp.concatenate(
        [xi[:, :1], xi, jnp.broadcast_to(xi[:, -1:], (b_total, 5))], axis=1)
    embh = _sc_gather(params["embedding"], idx1.reshape(b_total * _LH), 448)

    gp = params["gate"]
    top_idx = _gate_route(
        embh, gw[None], gp["cb"][None, :], gp["g"][None, :],
        gp["be"][None, :], gp["w1"].T, gp["b1"][None, :],
        gp["w2"].T, gp["b2"][None, :])

    nblk = b_total // _CAP + _E
    p = nblk * _CAP
    oh = (top_idx[:, None] == jnp.arange(_E, dtype=jnp.int32)).astype(jnp.int32)
    rank = jnp.sum((jnp.cumsum(oh, axis=0) - oh) * oh, axis=1)
    counts = jnp.sum(oh, axis=0)
    blocks_e = (counts + _CAP - 1) // _CAP
    start_blk = jnp.concatenate(
        [jnp.zeros((1,), jnp.int32), jnp.cumsum(blocks_e)[:-1]])
    dest = start_blk[top_idx] * _CAP + rank
    src = jnp.zeros((p,), jnp.int32).at[dest].set(
        jnp.arange(b_total, dtype=jnp.int32))
    gids = jnp.arange(nblk, dtype=jnp.int32)
    bexp = jnp.sum(gids[:, None] >= start_blk[None, :], axis=1,
                   dtype=jnp.int32) - 1
    nused = jnp.sum(blocks_e, dtype=jnp.int32)[None]

    xp_idx = (src[:, None] * _LH
              + jnp.arange(_LH, dtype=jnp.int32)[None, :]).reshape(p * _LH)
    emb_p = _sc_gather(embh, xp_idx, 448)
    out_p = _expert_blocks(
        emb_p, bexp, nused, *ws,
        params["fc1_w"].T, params["fc1_b"][None, :], f2w, f2b)

    out_rows = _sc_gather(out_p.reshape(p, 128), dest, 64)
    return out_rows[:, :2]


def kernel(x, params):
    gp = params["gate"]
    gw = jnp.transpose(gp["cw"], (2, 1, 0))
    ws1, ws2, ws3, bs1, bs2, bs3 = [], [], [], [], [], []
    for ep in params["experts"]:
        w1, b1 = _fold_conv(ep["w1"], ep["b1"], ep["g1"], ep["be1"])
        w2, b2 = _fold_conv(ep["w2"], ep["b2"], ep["g2"], ep["be2"])
        w3, b3 = _fold_conv(ep["w3"], ep["b3"], ep["g3"], ep["be3"])
        ws1.append(w1); ws2.append(w2); ws3.append(w3)
        bs1.append(b1); bs2.append(b2); bs3.append(b3)
    f2w = jnp.zeros((64, 16), jnp.float32).at[:, :2].set(params["fc2_w"].T)
    f2b = jnp.zeros((1, 16), jnp.float32).at[:, :2].set(params["fc2_b"][None])
    ws = (jnp.stack(ws1), jnp.stack(bs1), jnp.stack(ws2), jnp.stack(bs2),
          jnp.stack(ws3), jnp.stack(bs3))
    wts = (gw, ws, f2w, f2b)
    half = x.shape[0] // 2
    o0 = _run_half(x[:half], params, wts)
    o1 = _run_half(x[half:], params, wts)
    return jnp.concatenate([o0, o1], axis=0)
